# baseline (speedup 1.0000x reference)
import jax
import jax.numpy as jnp
from jax.experimental import pallas as pl
from jax.experimental.pallas import tpu as pltpu

_VOCAB = 8192
_DIM = 32
_COMMIT = 0.25
_BM = 1024
_BK = 2048
_NKB = _VOCAB // _BK


def _vq_body(x_ref, w_ref, x2_ref, w2_ref, idx_ref, qst_ref, mse_ref):
    x = x_ref[0]
    x2 = x2_ref[...]
    kiota = jax.lax.broadcasted_iota(jnp.int32, (_BM, _BK), 1)

    def dist_step(kb, carry):
        best, bidx = carry
        w_blk = w_ref[pl.ds(kb * _BK, _BK), :]
        w2_blk = w2_ref[:, pl.ds(kb * _BK, _BK)]
        xw = jax.lax.dot_general(
            x, w_blk, (((1,), (1,)), ((), ())),
            preferred_element_type=jnp.float32)
        d2 = (x2 + w2_blk) - 2.0 * xw
        s = jnp.sqrt(jnp.maximum(d2, 0.0))
        lmin = jnp.min(s, axis=1, keepdims=True)
        lidx = jnp.min(jnp.where(s == lmin, kiota, _BK),
                       axis=1, keepdims=True)
        upd = lmin < best
        return (jnp.where(upd, lmin, best),
                jnp.where(upd, lidx + kb * _BK, bidx))

    best0 = jnp.full((_BM, 1), jnp.inf, dtype=jnp.float32)
    bidx0 = jnp.zeros((_BM, 1), dtype=jnp.int32)
    best, bidx = jax.lax.fori_loop(0, _NKB, dist_step, (best0, bidx0))

    idx_ref[...] = bidx
    mse_ref[0] = jnp.sum(best * best, axis=0, keepdims=True)

    def gather_step(kb, q):
        w_blk = w_ref[pl.ds(kb * _BK, _BK), :]
        oh = (kiota == (bidx - kb * _BK)).astype(jnp.float32)
        return q + jax.lax.dot_general(
            oh, w_blk, (((1,), (0,)), ((), ())),
            preferred_element_type=jnp.float32)

    q = jax.lax.fori_loop(0, _NKB, gather_step,
                          jnp.zeros((_BM, _DIM), dtype=jnp.float32))
    qst_ref[0] = x + (q - x)


@jax.jit
def kernel(x, W):
    B, N, D = x.shape
    M = B * N
    x2 = jnp.sum(x * x, axis=-1, keepdims=True)
    w2 = jnp.sum(W * W, axis=-1)[None, :]
    x2f = x2.reshape(M, 1)

    grid = (M // _BM,)
    idx_flat, qst, mse_part = pl.pallas_call(
        _vq_body,
        grid=grid,
        in_specs=[
            pl.BlockSpec((1, _BM, D), lambda i: (i, 0, 0)),
            pl.BlockSpec((_VOCAB, D), lambda i: (0, 0)),
            pl.BlockSpec((_BM, 1), lambda i: (i, 0)),
            pl.BlockSpec((1, _VOCAB), lambda i: (0, 0)),
        ],
        out_specs=[
            pl.BlockSpec((_BM, 1), lambda i: (i, 0)),
            pl.BlockSpec((1, _BM, D), lambda i: (i, 0, 0)),
            pl.BlockSpec((1, 1, 1), lambda i: (i, 0, 0)),
        ],
        out_shape=[
            jax.ShapeDtypeStruct((M, 1), jnp.int32),
            jax.ShapeDtypeStruct((B, N, D), jnp.float32),
            jax.ShapeDtypeStruct((grid[0], 1, 1), jnp.float32),
        ],
    )(x.reshape(B, N, D), W, x2f, w2)

    mse = jnp.sum(mse_part) / (M * D)
    loss = mse + _COMMIT * mse
    return (qst, loss, mse, idx_flat.reshape(B, N))


# speedup vs baseline: 1.0705x; 1.0705x over previous
import functools

import jax
import jax.numpy as jnp
from jax import lax
from jax.experimental import pallas as pl
from jax.experimental.pallas import tpu as pltpu
from jax.experimental.pallas import tpu_sc as plsc

_VOCAB = 8192
_DIM = 32
_COMMIT = 0.25
_BM = 1024
_BK = 2048
_NKB = _VOCAB // _BK


def _vq_body(x_ref, w_ref, x2_ref, w2_ref, idx_ref, mse_ref):
    x = x_ref[0]
    x2 = x2_ref[...]
    kiota = jax.lax.broadcasted_iota(jnp.int32, (_BM, _BK), 1)

    def dist_step(kb, carry):
        best, bidx = carry
        w_blk = w_ref[pl.ds(kb * _BK, _BK), :]
        w2_blk = w2_ref[:, pl.ds(kb * _BK, _BK)]
        xw = jax.lax.dot_general(
            x, w_blk, (((1,), (1,)), ((), ())),
            preferred_element_type=jnp.float32)
        d2 = (x2 + w2_blk) - 2.0 * xw
        s = jnp.sqrt(jnp.maximum(d2, 0.0))
        lmin = jnp.min(s, axis=1, keepdims=True)
        lidx = jnp.min(jnp.where(s == lmin, kiota, _BK),
                       axis=1, keepdims=True)
        upd = lmin < best
        return (jnp.where(upd, lmin, best),
                jnp.where(upd, lidx + kb * _BK, bidx))

    best0 = jnp.full((_BM, 1), jnp.inf, dtype=jnp.float32)
    bidx0 = jnp.zeros((_BM, 1), dtype=jnp.int32)
    best, bidx = jax.lax.fori_loop(0, _NKB, dist_step, (best0, bidx0))

    idx_ref[...] = bidx
    mse_ref[0] = jnp.sum(best * best, axis=0, keepdims=True)


def _tc_call(x, W, x2f, w2, M, B, N, D):
    grid = (M // _BM,)
    return pl.pallas_call(
        _vq_body,
        grid=grid,
        in_specs=[
            pl.BlockSpec((1, _BM, D), lambda i: (i, 0, 0)),
            pl.BlockSpec((_VOCAB, D), lambda i: (0, 0)),
            pl.BlockSpec((_BM, 1), lambda i: (i, 0)),
            pl.BlockSpec((1, _VOCAB), lambda i: (0, 0)),
        ],
        out_specs=[
            pl.BlockSpec((_BM, 1), lambda i: (i, 0)),
            pl.BlockSpec((1, 1, 1), lambda i: (i, 0, 0)),
        ],
        out_shape=[
            jax.ShapeDtypeStruct((M, 1), jnp.int32),
            jax.ShapeDtypeStruct((grid[0], 1, 1), jnp.float32),
        ],
    )(x, W, x2f, w2)


_PAD = 128


def _make_sc_gather(M, D):
    info = plsc.get_sparse_core_info()
    NC, NS = info.num_cores, info.num_subcores
    NW = NC * NS
    b_per_w = M // NW
    mesh = plsc.VectorSubcoreMesh(core_axis_name="c", subcore_axis_name="s")

    @functools.partial(
        pl.kernel, mesh=mesh,
        out_type=jax.ShapeDtypeStruct((M, _PAD), jnp.float32),
        scratch_types=[
            pltpu.VMEM((b_per_w,), jnp.int32),
            pltpu.VMEM((b_per_w, _PAD), jnp.float32),
            pltpu.VMEM((b_per_w, D), jnp.float32),
            pltpu.SemaphoreType.DMA,
        ],
    )
    def sc_gather(wpad_hbm, idx_hbm, x_hbm, out_hbm, idx_v, rows_v, x_v, sem):
        wid = lax.axis_index("s") * NC + lax.axis_index("c")
        base = wid * b_per_w
        pltpu.sync_copy(idx_hbm.at[pl.ds(base, b_per_w)], idx_v)
        pltpu.async_copy(wpad_hbm.at[idx_v], rows_v, sem).wait()
        pltpu.sync_copy(x_hbm.at[pl.ds(base, b_per_w)], x_v)

        def body(r, c):
            for j in range(D // 16):
                xv = x_v[r, pl.ds(j * 16, 16)]
                qv = rows_v[r, pl.ds(j * 16, 16)]
                rows_v[r, pl.ds(j * 16, 16)] = xv + (qv - xv)
            return c

        jax.lax.fori_loop(0, b_per_w, body, 0)
        pltpu.sync_copy(rows_v, out_hbm.at[pl.ds(base, b_per_w)])

    return sc_gather


@jax.jit
def kernel(x, W):
    B, N, D = x.shape
    M = B * N
    x2 = jnp.sum(x * x, axis=-1, keepdims=True)
    w2 = jnp.sum(W * W, axis=-1)[None, :]
    x2f = x2.reshape(M, 1)

    idx_flat, mse_part = _tc_call(x, W, x2f, w2, M, B, N, D)

    xf = x.reshape(M, D)
    wpad = jnp.pad(W, ((0, 0), (0, _PAD - D)))
    qst_pad = _make_sc_gather(M, D)(wpad, idx_flat.reshape(M), xf)

    mse = jnp.sum(mse_part) / (M * D)
    loss = mse + _COMMIT * mse
    return (qst_pad[:, :D].reshape(B, N, D), loss, mse, idx_flat.reshape(B, N))


# speedup vs baseline: 1.1538x; 1.0778x over previous
import functools

import jax
import jax.numpy as jnp
from jax import lax
from jax.experimental import pallas as pl
from jax.experimental.pallas import tpu as pltpu
from jax.experimental.pallas import tpu_sc as plsc

_VOCAB = 8192
_DIM = 32
_COMMIT = 0.25
_BM = 1024
_BK = 4096
_NKB = _VOCAB // _BK


def _vq_body(x_ref, w_ref, x2_ref, w2_ref, idx_ref, mse_ref):
    x = x_ref[0]
    x2 = x2_ref[...]
    kiota = jax.lax.broadcasted_iota(jnp.int32, (_BM, _BK), 1)

    def dist_step(kb, carry):
        best, bidx = carry
        w_blk = w_ref[pl.ds(kb * _BK, _BK), :]
        w2_blk = w2_ref[:, pl.ds(kb * _BK, _BK)]
        xw = jax.lax.dot_general(
            x, w_blk, (((1,), (1,)), ((), ())),
            preferred_element_type=jnp.float32)
        d2 = (x2 + w2_blk) + xw
        s = jnp.sqrt(jnp.maximum(d2, 0.0))
        lmin = jnp.min(s, axis=1, keepdims=True)
        lidx = jnp.min(jnp.where(s == lmin, kiota, _BK),
                       axis=1, keepdims=True)
        upd = lmin < best
        return (jnp.where(upd, lmin, best),
                jnp.where(upd, lidx + kb * _BK, bidx))

    best0 = jnp.full((_BM, 1), jnp.inf, dtype=jnp.float32)
    bidx0 = jnp.zeros((_BM, 1), dtype=jnp.int32)
    best, bidx = jax.lax.fori_loop(0, _NKB, dist_step, (best0, bidx0))

    idx_ref[...] = bidx
    mse_ref[0] = jnp.sum(best * best, axis=0, keepdims=True)


def _tc_call(x, W, x2f, w2, M, B, N, D):
    grid = (M // _BM,)
    return pl.pallas_call(
        _vq_body,
        grid=grid,
        in_specs=[
            pl.BlockSpec((1, _BM, D), lambda i: (i, 0, 0)),
            pl.BlockSpec((_VOCAB, D), lambda i: (0, 0)),
            pl.BlockSpec((_BM, 1), lambda i: (i, 0)),
            pl.BlockSpec((1, _VOCAB), lambda i: (0, 0)),
        ],
        out_specs=[
            pl.BlockSpec((_BM, 1), lambda i: (i, 0)),
            pl.BlockSpec((1, 1, 1), lambda i: (i, 0, 0)),
        ],
        out_shape=[
            jax.ShapeDtypeStruct((M, 1), jnp.int32),
            jax.ShapeDtypeStruct((grid[0], 1, 1), jnp.float32),
        ],
    )(x, W, x2f, w2)


_PAD = 128


def _make_sc_gather(M, D):
    info = plsc.get_sparse_core_info()
    NC, NS = info.num_cores, info.num_subcores
    NW = NC * NS
    b_per_w = M // NW
    mesh = plsc.VectorSubcoreMesh(core_axis_name="c", subcore_axis_name="s")

    @functools.partial(
        pl.kernel, mesh=mesh,
        out_type=jax.ShapeDtypeStruct((M, _PAD), jnp.float32),
        scratch_types=[
            pltpu.VMEM((b_per_w,), jnp.int32),
            pltpu.VMEM((b_per_w, _PAD), jnp.float32),
            pltpu.VMEM((b_per_w, D), jnp.float32),
            pltpu.SemaphoreType.DMA,
        ],
    )
    def sc_gather(wpad_hbm, idx_hbm, x_hbm, out_hbm, idx_v, rows_v, x_v, sem):
        wid = lax.axis_index("s") * NC + lax.axis_index("c")
        base = wid * b_per_w
        pltpu.sync_copy(idx_hbm.at[pl.ds(base, b_per_w)], idx_v)
        pltpu.async_copy(wpad_hbm.at[idx_v], rows_v, sem).wait()
        pltpu.sync_copy(x_hbm.at[pl.ds(base, b_per_w)], x_v)

        def body(r, c):
            for j in range(D // 16):
                xv = x_v[r, pl.ds(j * 16, 16)]
                qv = rows_v[r, pl.ds(j * 16, 16)]
                rows_v[r, pl.ds(j * 16, 16)] = xv + (qv - xv)
            return c

        jax.lax.fori_loop(0, b_per_w, body, 0)
        pltpu.sync_copy(rows_v, out_hbm.at[pl.ds(base, b_per_w)])

    return sc_gather


@jax.jit
def kernel(x, W):
    B, N, D = x.shape
    M = B * N
    x2 = jnp.sum(x * x, axis=-1, keepdims=True)
    w2 = jnp.sum(W * W, axis=-1)[None, :]
    x2f = x2.reshape(M, 1)
    xm2 = x * (-2.0)

    idx_flat, mse_part = _tc_call(xm2, W, x2f, w2, M, B, N, D)

    xf = x.reshape(M, D)
    wpad = jnp.pad(W, ((0, 0), (0, _PAD - D)))
    qst_pad = _make_sc_gather(M, D)(wpad, idx_flat.reshape(M), xf)

    mse = jnp.sum(mse_part) / (M * D)
    loss = mse + _COMMIT * mse
    return (qst_pad[:, :D].reshape(B, N, D), loss, mse, idx_flat.reshape(B, N))


# speedup vs baseline: 1.3121x; 1.1371x over previous
import functools

import jax
import jax.numpy as jnp
from jax import lax
from jax.experimental import pallas as pl
from jax.experimental.pallas import tpu as pltpu
from jax.experimental.pallas import tpu_sc as plsc

_VOCAB = 8192
_DIM = 32
_COMMIT = 0.25
_BM = 1024
_BK = 4096
_NKB = _VOCAB // _BK


def _vq_body(x_ref, w_ref, x2_ref, w2_ref, idx_ref, mse_ref):
    x = x_ref[0]
    x2 = x2_ref[...]
    kiota = jax.lax.broadcasted_iota(jnp.int32, (_BM, _BK), 1)

    def dist_step(kb, carry):
        best, bidx = carry
        w_blk = w_ref[pl.ds(kb * _BK, _BK), :]
        w2_blk = w2_ref[:, pl.ds(kb * _BK, _BK)]
        xw = jax.lax.dot_general(
            x, w_blk, (((1,), (1,)), ((), ())),
            preferred_element_type=jnp.float32)
        d2 = (x2 + w2_blk) + xw
        d2c = jnp.maximum(d2, 0.0)
        s = jnp.where(d2c < 1.1754944e-38, 0.0,
                      d2c * jax.lax.rsqrt(d2c))
        lmin = jnp.min(s, axis=1, keepdims=True)
        lidx = jnp.min(jnp.where(s == lmin, kiota, _BK),
                       axis=1, keepdims=True)
        upd = lmin < best
        return (jnp.where(upd, lmin, best),
                jnp.where(upd, lidx + kb * _BK, bidx))

    best0 = jnp.full((_BM, 1), jnp.inf, dtype=jnp.float32)
    bidx0 = jnp.zeros((_BM, 1), dtype=jnp.int32)
    best, bidx = jax.lax.fori_loop(0, _NKB, dist_step, (best0, bidx0))

    idx_ref[...] = bidx
    mse_ref[0] = jnp.sum(best * best, axis=0, keepdims=True)


def _tc_call(x, W, x2f, w2, M, B, N, D):
    grid = (M // _BM,)
    return pl.pallas_call(
        _vq_body,
        grid=grid,
        in_specs=[
            pl.BlockSpec((1, _BM, D), lambda i: (i, 0, 0)),
            pl.BlockSpec((_VOCAB, D), lambda i: (0, 0)),
            pl.BlockSpec((_BM, 1), lambda i: (i, 0)),
            pl.BlockSpec((1, _VOCAB), lambda i: (0, 0)),
        ],
        out_specs=[
            pl.BlockSpec((_BM, 1), lambda i: (i, 0)),
            pl.BlockSpec((1, 1, 1), lambda i: (i, 0, 0)),
        ],
        out_shape=[
            jax.ShapeDtypeStruct((M, 1), jnp.int32),
            jax.ShapeDtypeStruct((grid[0], 1, 1), jnp.float32),
        ],
    )(x, W, x2f, w2)


_PAD = 128


def _make_sc_gather(M, D):
    info = plsc.get_sparse_core_info()
    NC, NS = info.num_cores, info.num_subcores
    NW = NC * NS
    b_per_w = M // NW
    mesh = plsc.VectorSubcoreMesh(core_axis_name="c", subcore_axis_name="s")

    @functools.partial(
        pl.kernel, mesh=mesh,
        out_type=jax.ShapeDtypeStruct((M, _PAD), jnp.float32),
        scratch_types=[
            pltpu.VMEM((b_per_w,), jnp.int32),
            pltpu.VMEM((b_per_w, _PAD), jnp.float32),
            pltpu.VMEM((b_per_w, D), jnp.float32),
            pltpu.SemaphoreType.DMA,
        ],
    )
    def sc_gather(wpad_hbm, idx_hbm, x_hbm, out_hbm, idx_v, rows_v, x_v, sem):
        wid = lax.axis_index("s") * NC + lax.axis_index("c")
        base = wid * b_per_w
        pltpu.sync_copy(idx_hbm.at[pl.ds(base, b_per_w)], idx_v)
        pltpu.async_copy(wpad_hbm.at[idx_v], rows_v, sem).wait()
        pltpu.sync_copy(x_hbm.at[pl.ds(base, b_per_w)], x_v)

        def body(r, c):
            for j in range(D // 16):
                xv = x_v[r, pl.ds(j * 16, 16)]
                qv = rows_v[r, pl.ds(j * 16, 16)]
                rows_v[r, pl.ds(j * 16, 16)] = xv + (qv - xv)
            return c

        jax.lax.fori_loop(0, b_per_w, body, 0)
        pltpu.sync_copy(rows_v, out_hbm.at[pl.ds(base, b_per_w)])

    return sc_gather


@jax.jit
def kernel(x, W):
    B, N, D = x.shape
    M = B * N
    x2 = jnp.sum(x * x, axis=-1, keepdims=True)
    w2 = jnp.sum(W * W, axis=-1)[None, :]
    x2f = x2.reshape(M, 1)
    xm2 = x * (-2.0)

    idx_flat, mse_part = _tc_call(xm2, W, x2f, w2, M, B, N, D)

    xf = x.reshape(M, D)
    wpad = jnp.pad(W, ((0, 0), (0, _PAD - D)))
    qst_pad = _make_sc_gather(M, D)(wpad, idx_flat.reshape(M), xf)

    mse = jnp.sum(mse_part) / (M * D)
    loss = mse + _COMMIT * mse
    return (qst_pad[:, :D].reshape(B, N, D), loss, mse, idx_flat.reshape(B, N))
